# Initial kernel scaffold; baseline (speedup 1.0000x reference)
import functools

import jax
import jax.numpy as jnp
from jax import lax
from jax.experimental import pallas as pl
from jax.experimental.pallas import tpu as pltpu
from jax.experimental.pallas import tpu_sc as plsc


def _make_gather(n_total, vocab, dim, n_cores, n_subcores):
  n_workers = n_cores * n_subcores
  rows_per_gather = 128
  gathers_per_group = 8
  group_rows = rows_per_gather * gathers_per_group
  rows_per_worker = n_total // n_workers
  assert rows_per_worker % group_rows == 0
  n_groups = rows_per_worker // group_rows

  mesh = plsc.VectorSubcoreMesh(core_axis_name="c", subcore_axis_name="s")

  @functools.partial(
      pl.kernel,
      mesh=mesh,
      out_type=jax.ShapeDtypeStruct((n_total, dim), jnp.float32),
      scratch_types=[
          pltpu.VMEM((gathers_per_group, rows_per_gather), jnp.int32),
          pltpu.VMEM((group_rows, dim), jnp.float32),
          pltpu.SemaphoreType.DMA,
      ],
  )
  def emb(ids_hbm, table_hbm, out_hbm, idx_v, rows_v, sem):
    wid = lax.axis_index("s") * n_cores + lax.axis_index("c")
    idxrow_base = wid * (rows_per_worker // rows_per_gather)
    out_base = wid * rows_per_worker

    def step(g, carry):
      pltpu.sync_copy(
          ids_hbm.at[pl.ds(idxrow_base + g * gathers_per_group,
                           gathers_per_group)],
          idx_v)
      copies = []
      for j in range(gathers_per_group):
        copies.append(
            pltpu.async_copy(
                table_hbm.at[idx_v.at[j]],
                rows_v.at[pl.ds(j * rows_per_gather, rows_per_gather)],
                sem))
      for c in copies:
        c.wait()
      pltpu.sync_copy(rows_v,
                      out_hbm.at[pl.ds(out_base + g * group_rows, group_rows)])
      return carry

    lax.fori_loop(0, n_groups, step, 0)

  return emb


def kernel(input_ids, table):
  batch, hist = input_ids.shape
  vocab, dim = table.shape
  n_total = batch * hist
  info = plsc.get_sparse_core_info()
  ids_flat = input_ids.reshape(n_total // 128, 128).astype(jnp.int32)
  emb = _make_gather(n_total, vocab, dim, info.num_cores, info.num_subcores)
  out = emb(ids_flat, table)
  return out.reshape(batch, hist, dim)


# speedup vs baseline: 1.8449x; 1.8449x over previous
import functools

import jax
import jax.numpy as jnp
from jax import lax
from jax.experimental import pallas as pl
from jax.experimental.pallas import tpu as pltpu
from jax.experimental.pallas import tpu_sc as plsc


def _make_gather(n_total, vocab, dim, n_cores, n_subcores):
  n_workers = n_cores * n_subcores
  rows_per_gather = 128
  gathers_per_group = 8
  group_rows = rows_per_gather * gathers_per_group
  rows_per_worker = n_total // n_workers
  assert rows_per_worker % group_rows == 0
  n_groups = rows_per_worker // group_rows

  mesh = plsc.VectorSubcoreMesh(core_axis_name="c", subcore_axis_name="s")

  @functools.partial(
      pl.kernel,
      mesh=mesh,
      compiler_params=pltpu.CompilerParams(use_tc_tiling_on_sc=False),
      out_type=jax.ShapeDtypeStruct((n_total, dim), jnp.float32),
      scratch_types=[
          pltpu.VMEM((gathers_per_group, rows_per_gather), jnp.int32),
          pltpu.VMEM((group_rows, dim), jnp.float32),
          pltpu.SemaphoreType.DMA,
      ],
  )
  def emb(ids_hbm, table_hbm, out_hbm, idx_v, rows_v, sem):
    wid = lax.axis_index("s") * n_cores + lax.axis_index("c")
    idxrow_base = wid * (rows_per_worker // rows_per_gather)
    out_base = wid * rows_per_worker

    def step(g, carry):
      pltpu.sync_copy(
          ids_hbm.at[pl.ds(idxrow_base + g * gathers_per_group,
                           gathers_per_group)],
          idx_v)
      copies = []
      for j in range(gathers_per_group):
        copies.append(
            pltpu.async_copy(
                table_hbm.at[idx_v.at[j]],
                rows_v.at[pl.ds(j * rows_per_gather, rows_per_gather)],
                sem))
      for c in copies:
        c.wait()
      pltpu.sync_copy(rows_v,
                      out_hbm.at[pl.ds(out_base + g * group_rows, group_rows)])
      return carry

    lax.fori_loop(0, n_groups, step, 0)

  return emb


def kernel(input_ids, table):
  batch, hist = input_ids.shape
  vocab, dim = table.shape
  n_total = batch * hist
  info = plsc.get_sparse_core_info()
  ids_flat = input_ids.reshape(n_total // 128, 128).astype(jnp.int32)
  emb = _make_gather(n_total, vocab, dim, info.num_cores, info.num_subcores)
  out = emb(ids_flat, table)
  return out.reshape(batch, hist, dim)
